# baseline (speedup 1.0000x reference)
import functools

import jax
import jax.numpy as jnp
import numpy as np
from jax.experimental import pallas as pl
from jax.experimental.pallas import tpu as pltpu


def _phase_a_kernel(x_ref, w1_ref, b1_ref, w2_ref, b2_ref, adj_ref,
                    s2_out_ref, part_ref, sup_s, s2_s, *, bm, nb, n):
    i = pl.program_id(0)

    @pl.when(i == 0)
    def _():
        sup_s[...] = jnp.dot(x_ref[...], w1_ref[...],
                             preferred_element_type=jnp.float32)
        s2_s[...] = jnp.zeros_like(s2_s)

    h = jnp.dot(adj_ref[...], sup_s[...],
                preferred_element_type=jnp.float32)
    h = jnp.maximum(h + b1_ref[...], 0.0)
    s2 = jnp.dot(h, w2_ref[...], preferred_element_type=jnp.float32)
    row = jax.lax.broadcasted_iota(jnp.int32, s2.shape, 0)
    s2 = jnp.where(row < n - i * bm, s2, 0.0)

    @pl.when(i < nb - 1)
    def _():
        s2_s[pl.ds(i * bm, bm), :] = s2

    part_ref[...] = jnp.dot(adj_ref[...], s2_s[...],
                            preferred_element_type=jnp.float32) + b2_ref[...]
    s2_out_ref[...] = s2


def _phase_b_kernel(ti_ref, tk_ref, fs_ref, fe_ref, adj_ref, s2_ref,
                    part_ref, o_ref, *, bm):
    t = pl.program_id(0)
    s2_blk = s2_ref[pl.ds(tk_ref[t] * bm, bm), :]
    contrib = jnp.dot(adj_ref[...], s2_blk,
                      preferred_element_type=jnp.float32)

    @pl.when(fs_ref[t] == 1)
    def _():
        o_ref[...] = part_ref[...]

    o_ref[...] += contrib

    @pl.when(fe_ref[t] == 1)
    def _():
        logits = o_ref[...]
        m = jnp.max(logits, axis=1, keepdims=True)
        z = logits - m
        lse = jnp.log(jnp.sum(jnp.exp(z), axis=1, keepdims=True))
        o_ref[...] = z - lse


def kernel(x, adj, W1, b1, W2, b2):
    n, f_in = x.shape
    h_dim = W1.shape[1]
    c_dim = W2.shape[1]
    bm = 512
    nb = -(-n // bm)

    b1_2d = b1.reshape(1, h_dim)
    b2_2d = b2.reshape(1, c_dim)

    s2_hbm, partial = pl.pallas_call(
        functools.partial(_phase_a_kernel, bm=bm, nb=nb, n=n),
        grid=(nb,),
        in_specs=[
            pl.BlockSpec((n, f_in), lambda i: (0, 0)),
            pl.BlockSpec((f_in, h_dim), lambda i: (0, 0)),
            pl.BlockSpec((1, h_dim), lambda i: (0, 0)),
            pl.BlockSpec((h_dim, c_dim), lambda i: (0, 0)),
            pl.BlockSpec((1, c_dim), lambda i: (0, 0)),
            pl.BlockSpec((bm, n), lambda i: (i, 0)),
        ],
        out_specs=[
            pl.BlockSpec((bm, c_dim), lambda i: (i, 0)),
            pl.BlockSpec((bm, c_dim), lambda i: (i, 0)),
        ],
        out_shape=[
            jax.ShapeDtypeStruct((nb * bm, c_dim), jnp.float32),
            jax.ShapeDtypeStruct((n, c_dim), jnp.float32),
        ],
        scratch_shapes=[
            pltpu.VMEM((n, h_dim), jnp.float32),
            pltpu.VMEM((n, c_dim), jnp.float32),
        ],
        compiler_params=pltpu.CompilerParams(
            dimension_semantics=("arbitrary",)),
    )(x, W1, b1_2d, W2, b2_2d, adj)

    ti, tk, fs, fe = [], [], [], []
    for i in range(nb):
        ks = list(range(i + 1, nb)) if i < nb - 1 else [nb - 1]
        for j, k in enumerate(ks):
            ti.append(i)
            tk.append(k)
            fs.append(1 if j == 0 else 0)
            fe.append(1 if k == ks[-1] else 0)
    ti = jnp.asarray(np.array(ti, dtype=np.int32))
    tk = jnp.asarray(np.array(tk, dtype=np.int32))
    fs = jnp.asarray(np.array(fs, dtype=np.int32))
    fe = jnp.asarray(np.array(fe, dtype=np.int32))
    n_tiles = ti.shape[0]

    out = pl.pallas_call(
        functools.partial(_phase_b_kernel, bm=bm),
        grid_spec=pltpu.PrefetchScalarGridSpec(
            num_scalar_prefetch=4,
            grid=(n_tiles,),
            in_specs=[
                pl.BlockSpec((bm, bm),
                             lambda t, ti, tk, fs, fe: (ti[t], tk[t])),
                pl.BlockSpec((nb * bm, c_dim),
                             lambda t, ti, tk, fs, fe: (0, 0)),
                pl.BlockSpec((bm, c_dim),
                             lambda t, ti, tk, fs, fe: (ti[t], 0)),
            ],
            out_specs=pl.BlockSpec((bm, c_dim),
                                   lambda t, ti, tk, fs, fe: (ti[t], 0)),
        ),
        out_shape=jax.ShapeDtypeStruct((n, c_dim), jnp.float32),
        compiler_params=pltpu.CompilerParams(
            dimension_semantics=("arbitrary",)),
    )(ti, tk, fs, fe, adj, s2_hbm, partial)

    return out


# speedup vs baseline: 1.1416x; 1.1416x over previous
import functools

import jax
import jax.numpy as jnp
import numpy as np
from jax.experimental import pallas as pl
from jax.experimental.pallas import tpu as pltpu


def _phase_a_kernel(x_ref, w1_ref, b1_ref, w2_ref, b2_ref, adj_ref,
                    s2_out_ref, part_ref, sup_s, s2_s, prev_s,
                    *, bm, nb, n):
    i = pl.program_id(0)

    @pl.when(i == 0)
    def _():
        sup_s[...] = jnp.dot(x_ref[...], w1_ref[...],
                             preferred_element_type=jnp.float32)
        s2_s[...] = jnp.zeros_like(s2_s)

    h = jnp.dot(adj_ref[...], sup_s[...],
                preferred_element_type=jnp.float32)
    h = jnp.maximum(h + b1_ref[...], 0.0)
    s2 = jnp.dot(h, w2_ref[...], preferred_element_type=jnp.float32)
    row = jax.lax.broadcasted_iota(jnp.int32, s2.shape, 0)
    s2 = jnp.where(row < n - i * bm, s2, 0.0)

    @pl.when((i % 2 == 1) & (i < nb - 1))
    def _():
        s2_s[pl.ds((i - 1) * bm, bm), :] = prev_s[...]
        s2_s[pl.ds(i * bm, bm), :] = s2

    part_ref[...] = jnp.dot(adj_ref[...], s2_s[...],
                            preferred_element_type=jnp.float32) + b2_ref[...]
    prev_s[...] = s2
    s2_out_ref[...] = s2


def _phase_b_kernel(ti_ref, tk_ref, fs_ref, fe_ref, adj_ref, s2_ref,
                    part_ref, o_ref, *, wb):
    t = pl.program_id(0)
    s2_blk = s2_ref[pl.ds(tk_ref[t] * wb, wb), :]
    contrib = jnp.dot(adj_ref[...], s2_blk,
                      preferred_element_type=jnp.float32)

    @pl.when(fs_ref[t] == 1)
    def _():
        o_ref[...] = part_ref[...]

    o_ref[...] += contrib

    @pl.when(fe_ref[t] == 1)
    def _():
        logits = o_ref[...]
        m = jnp.max(logits, axis=1, keepdims=True)
        z = logits - m
        lse = jnp.log(jnp.sum(jnp.exp(z), axis=1, keepdims=True))
        o_ref[...] = z - lse


def kernel(x, adj, W1, b1, W2, b2):
    n, f_in = x.shape
    h_dim = W1.shape[1]
    c_dim = W2.shape[1]
    bm = 512
    wb = 1024
    nb = -(-n // bm)
    nk = -(-n // wb)

    b1_2d = b1.reshape(1, h_dim)
    b2_2d = b2.reshape(1, c_dim)

    s2_hbm, partial = pl.pallas_call(
        functools.partial(_phase_a_kernel, bm=bm, nb=nb, n=n),
        grid=(nb,),
        in_specs=[
            pl.BlockSpec((n, f_in), lambda i: (0, 0)),
            pl.BlockSpec((f_in, h_dim), lambda i: (0, 0)),
            pl.BlockSpec((1, h_dim), lambda i: (0, 0)),
            pl.BlockSpec((h_dim, c_dim), lambda i: (0, 0)),
            pl.BlockSpec((1, c_dim), lambda i: (0, 0)),
            pl.BlockSpec((bm, n), lambda i: (i, 0)),
        ],
        out_specs=[
            pl.BlockSpec((bm, c_dim), lambda i: (i, 0)),
            pl.BlockSpec((bm, c_dim), lambda i: (i, 0)),
        ],
        out_shape=[
            jax.ShapeDtypeStruct((nb * bm, c_dim), jnp.float32),
            jax.ShapeDtypeStruct((n, c_dim), jnp.float32),
        ],
        scratch_shapes=[
            pltpu.VMEM((n, h_dim), jnp.float32),
            pltpu.VMEM((n, c_dim), jnp.float32),
            pltpu.VMEM((bm, c_dim), jnp.float32),
        ],
        compiler_params=pltpu.CompilerParams(
            dimension_semantics=("arbitrary",)),
    )(x, W1, b1_2d, W2, b2_2d, adj)

    ti, tk, fs, fe = [], [], [], []
    for i in range(nb):
        pb = 2 * ((i + 1) // 2) if i < nb - 1 else nb - 2
        ks = list(range(pb // 2, nk))
        for j, k in enumerate(ks):
            ti.append(i)
            tk.append(k)
            fs.append(1 if j == 0 else 0)
            fe.append(1 if k == ks[-1] else 0)
    ti = jnp.asarray(np.array(ti, dtype=np.int32))
    tk = jnp.asarray(np.array(tk, dtype=np.int32))
    fs = jnp.asarray(np.array(fs, dtype=np.int32))
    fe = jnp.asarray(np.array(fe, dtype=np.int32))
    n_tiles = ti.shape[0]

    out = pl.pallas_call(
        functools.partial(_phase_b_kernel, wb=wb),
        grid_spec=pltpu.PrefetchScalarGridSpec(
            num_scalar_prefetch=4,
            grid=(n_tiles,),
            in_specs=[
                pl.BlockSpec((bm, wb),
                             lambda t, ti, tk, fs, fe: (ti[t], tk[t])),
                pl.BlockSpec((nb * bm, c_dim),
                             lambda t, ti, tk, fs, fe: (0, 0)),
                pl.BlockSpec((bm, c_dim),
                             lambda t, ti, tk, fs, fe: (ti[t], 0)),
            ],
            out_specs=pl.BlockSpec((bm, c_dim),
                                   lambda t, ti, tk, fs, fe: (ti[t], 0)),
        ),
        out_shape=jax.ShapeDtypeStruct((n, c_dim), jnp.float32),
        compiler_params=pltpu.CompilerParams(
            dimension_semantics=("arbitrary",)),
    )(ti, tk, fs, fe, adj, s2_hbm, partial)

    return out


# speedup vs baseline: 1.1707x; 1.0255x over previous
import functools

import jax
import jax.numpy as jnp
import numpy as np
from jax.experimental import pallas as pl
from jax.experimental.pallas import tpu as pltpu


def _log_softmax(logits):
    m = jnp.max(logits, axis=1, keepdims=True)
    z = logits - m
    lse = jnp.log(jnp.sum(jnp.exp(z), axis=1, keepdims=True))
    return z - lse


def _call1_kernel(r_ref, p_ref, x_ref, w1_ref, b1_ref, w2_ref, b2_ref,
                  adj_ref, s2_out_ref, part_ref, out_ref,
                  sup_s, s2_s, prev_s, *, bm, nb, n):
    t = pl.program_id(0)
    i = r_ref[t]
    p = p_ref[t]

    @pl.when(t == 0)
    def _():
        sup_s[...] = jnp.dot(x_ref[...], w1_ref[...],
                             preferred_element_type=jnp.float32)
        s2_s[...] = jnp.zeros_like(s2_s)

    @pl.when(p == 0)
    def _():
        h = jnp.dot(adj_ref[...], sup_s[...],
                    preferred_element_type=jnp.float32)
        h = jnp.maximum(h + b1_ref[...], 0.0)
        s2 = jnp.dot(h, w2_ref[...], preferred_element_type=jnp.float32)
        row = jax.lax.broadcasted_iota(jnp.int32, s2.shape, 0)
        s2 = jnp.where(row < n - i * bm, s2, 0.0)

        @pl.when(i % 2 == 1)
        def _():
            s2_s[pl.ds((i - 1) * bm, bm), :] = prev_s[...]
            s2_s[pl.ds(i * bm, bm), :] = s2

        partial = jnp.dot(adj_ref[...], s2_s[pl.ds(0, n), :],
                          preferred_element_type=jnp.float32) + b2_ref[...]
        part_ref[...] = partial

        @pl.when(i == nb - 1)
        def _():
            out_ref[...] = _log_softmax(partial)

        prev_s[...] = s2
        s2_out_ref[...] = s2

    @pl.when(p == 1)
    def _():
        logits = jnp.dot(adj_ref[...], s2_s[pl.ds(0, n), :],
                         preferred_element_type=jnp.float32) + b2_ref[...]
        out_ref[...] = _log_softmax(logits)


def _tile_kernel(ti_ref, tk_ref, fs_ref, fe_ref, adj_ref, s2_ref,
                 part_ref, o_ref, *, wb, i0):
    t = pl.program_id(0)
    s2_blk = s2_ref[pl.ds(tk_ref[t] * wb, wb), :]
    contrib = jnp.dot(adj_ref[...], s2_blk,
                      preferred_element_type=jnp.float32)

    @pl.when(fs_ref[t] == 1)
    def _():
        o_ref[...] = part_ref[...]

    o_ref[...] += contrib

    @pl.when(fe_ref[t] == 1)
    def _():
        o_ref[...] = _log_softmax(o_ref[...])


def kernel(x, adj, W1, b1, W2, b2):
    n, f_in = x.shape
    h_dim = W1.shape[1]
    c_dim = W2.shape[1]
    bm = 512
    wb = 1024
    nb = -(-n // bm)
    nk = -(-n // wb)
    i0 = max(nb - 5, 0)

    b1_2d = b1.reshape(1, h_dim)
    b2_2d = b2.reshape(1, c_dim)

    rows = list(range(nb)) + list(range(i0))
    phases = [0] * nb + [1] * i0
    r_tab = jnp.asarray(np.array(rows, dtype=np.int32))
    p_tab = jnp.asarray(np.array(phases, dtype=np.int32))
    t_steps = len(rows)

    s2_hbm, partial, out1 = pl.pallas_call(
        functools.partial(_call1_kernel, bm=bm, nb=nb, n=n),
        grid_spec=pltpu.PrefetchScalarGridSpec(
            num_scalar_prefetch=2,
            grid=(t_steps,),
            in_specs=[
                pl.BlockSpec((n, f_in), lambda t, r, p: (0, 0)),
                pl.BlockSpec((f_in, h_dim), lambda t, r, p: (0, 0)),
                pl.BlockSpec((1, h_dim), lambda t, r, p: (0, 0)),
                pl.BlockSpec((h_dim, c_dim), lambda t, r, p: (0, 0)),
                pl.BlockSpec((1, c_dim), lambda t, r, p: (0, 0)),
                pl.BlockSpec((bm, n), lambda t, r, p: (r[t], 0)),
            ],
            out_specs=[
                pl.BlockSpec((bm, c_dim), lambda t, r, p: (r[t], 0)),
                pl.BlockSpec((bm, c_dim), lambda t, r, p: (r[t], 0)),
                pl.BlockSpec((bm, c_dim), lambda t, r, p: (r[t], 0)),
            ],
            scratch_shapes=[
                pltpu.VMEM((n, h_dim), jnp.float32),
                pltpu.VMEM((nb * bm, c_dim), jnp.float32),
                pltpu.VMEM((bm, c_dim), jnp.float32),
            ],
        ),
        out_shape=[
            jax.ShapeDtypeStruct((nb * bm, c_dim), jnp.float32),
            jax.ShapeDtypeStruct((n, c_dim), jnp.float32),
            jax.ShapeDtypeStruct((n, c_dim), jnp.float32),
        ],
        compiler_params=pltpu.CompilerParams(
            dimension_semantics=("arbitrary",)),
    )(r_tab, p_tab, x, W1, b1_2d, W2, b2_2d, adj)

    ti, tk, fs, fe = [], [], [], []
    for i in range(i0, nb - 1):
        pb = 2 * ((i + 1) // 2)
        ks = list(range(pb // 2, nk))
        for j, k in enumerate(ks):
            ti.append(i)
            tk.append(k)
            fs.append(1 if j == 0 else 0)
            fe.append(1 if k == ks[-1] else 0)
    ti_t = jnp.asarray(np.array(ti, dtype=np.int32))
    tk_t = jnp.asarray(np.array(tk, dtype=np.int32))
    fs_t = jnp.asarray(np.array(fs, dtype=np.int32))
    fe_t = jnp.asarray(np.array(fe, dtype=np.int32))
    n_tiles = ti_t.shape[0]

    out2 = pl.pallas_call(
        functools.partial(_tile_kernel, wb=wb, i0=i0),
        grid_spec=pltpu.PrefetchScalarGridSpec(
            num_scalar_prefetch=4,
            grid=(n_tiles,),
            in_specs=[
                pl.BlockSpec((bm, wb),
                             lambda t, ti, tk, fs, fe: (ti[t], tk[t])),
                pl.BlockSpec((nb * bm, c_dim),
                             lambda t, ti, tk, fs, fe: (0, 0)),
                pl.BlockSpec((bm, c_dim),
                             lambda t, ti, tk, fs, fe: (ti[t], 0)),
            ],
            out_specs=pl.BlockSpec((bm, c_dim),
                                   lambda t, ti, tk, fs, fe: (ti[t] - i0, 0)),
        ),
        out_shape=jax.ShapeDtypeStruct(((nb - 1 - i0) * bm, c_dim),
                                       jnp.float32),
        compiler_params=pltpu.CompilerParams(
            dimension_semantics=("arbitrary",)),
    )(ti_t, tk_t, fs_t, fe_t, adj, s2_hbm, partial)

    lo = i0 * bm
    hi = (nb - 1) * bm
    return jnp.concatenate([out1[:lo], out2, out1[hi:n]], axis=0)


# speedup vs baseline: 1.6437x; 1.4040x over previous
import functools

import jax
import jax.numpy as jnp
import numpy as np
from jax.experimental import pallas as pl
from jax.experimental.pallas import tpu as pltpu


def _log_softmax(logits):
    m = jnp.max(logits, axis=1, keepdims=True)
    z = logits - m
    lse = jnp.log(jnp.sum(jnp.exp(z), axis=1, keepdims=True))
    return z - lse


def _call1_kernel(x_ref, w1_ref, b1_ref, w2_ref, b2_ref,
                  adj_ref, s2_out_ref, part_ref, out_ref,
                  sup_s, s2_s, prev_s, *, bm, nb, n, i0):
    t = pl.program_id(0)
    i = jax.lax.rem(t, nb)
    p = jax.lax.div(t, nb)

    @pl.when(t == 0)
    def _():
        sup_s[...] = jnp.dot(x_ref[...], w1_ref[...],
                             preferred_element_type=jnp.float32)
        s2_s[...] = jnp.zeros_like(s2_s)

    @pl.when(p == 0)
    def _():
        h = jnp.dot(adj_ref[...], sup_s[...],
                    preferred_element_type=jnp.float32)
        h = jnp.maximum(h + b1_ref[...], 0.0)
        s2 = jnp.dot(h, w2_ref[...], preferred_element_type=jnp.float32)
        row = jax.lax.broadcasted_iota(jnp.int32, s2.shape, 0)
        s2 = jnp.where(row < n - i * bm, s2, 0.0)

        @pl.when(i % 2 == 1)
        def _():
            s2_s[pl.ds((i - 1) * bm, bm), :] = prev_s[...]
            s2_s[pl.ds(i * bm, bm), :] = s2

        @pl.when(i >= i0)
        def _():
            partial = jnp.dot(adj_ref[...], s2_s[pl.ds(0, n), :],
                              preferred_element_type=jnp.float32)
            partial = partial + b2_ref[...]
            part_ref[...] = partial

            @pl.when(i == nb - 1)
            def _():
                out_ref[...] = _log_softmax(partial)

        prev_s[...] = s2
        s2_out_ref[...] = s2

    @pl.when(p == 1)
    def _():
        logits = jnp.dot(adj_ref[...], s2_s[pl.ds(0, n), :],
                         preferred_element_type=jnp.float32) + b2_ref[...]
        out_ref[...] = _log_softmax(logits)


def _tile_kernel(ti_ref, tk_ref, fs_ref, fe_ref, adj_ref, s2_ref,
                 part_ref, o_ref, *, wb, i0):
    t = pl.program_id(0)
    s2_blk = s2_ref[pl.ds(tk_ref[t] * wb, wb), :]
    contrib = jnp.dot(adj_ref[...], s2_blk,
                      preferred_element_type=jnp.float32)

    @pl.when(fs_ref[t] == 1)
    def _():
        o_ref[...] = part_ref[...]

    o_ref[...] += contrib

    @pl.when(fe_ref[t] == 1)
    def _():
        o_ref[...] = _log_softmax(o_ref[...])


def kernel(x, adj, W1, b1, W2, b2):
    n, f_in = x.shape
    h_dim = W1.shape[1]
    c_dim = W2.shape[1]
    bm = 512
    wb = 1024
    nb = -(-n // bm)
    nk = -(-n // wb)
    i0 = max(nb - 5, 0)

    b1_2d = b1.reshape(1, h_dim)
    b2_2d = b2.reshape(1, c_dim)

    t_steps = nb + i0

    def row_map(t):
        return (jax.lax.rem(t, nb), 0)

    s2_hbm, partial, out1 = pl.pallas_call(
        functools.partial(_call1_kernel, bm=bm, nb=nb, n=n, i0=i0),
        grid=(t_steps,),
        in_specs=[
            pl.BlockSpec((n, f_in), lambda t: (0, 0)),
            pl.BlockSpec((f_in, h_dim), lambda t: (0, 0)),
            pl.BlockSpec((1, h_dim), lambda t: (0, 0)),
            pl.BlockSpec((h_dim, c_dim), lambda t: (0, 0)),
            pl.BlockSpec((1, c_dim), lambda t: (0, 0)),
            pl.BlockSpec((bm, n), row_map),
        ],
        out_specs=[
            pl.BlockSpec((bm, c_dim), row_map),
            pl.BlockSpec((bm, c_dim), row_map),
            pl.BlockSpec((bm, c_dim), row_map),
        ],
        scratch_shapes=[
            pltpu.VMEM((n, h_dim), jnp.float32),
            pltpu.VMEM((nb * bm, c_dim), jnp.float32),
            pltpu.VMEM((bm, c_dim), jnp.float32),
        ],
        out_shape=[
            jax.ShapeDtypeStruct((nb * bm, c_dim), jnp.float32),
            jax.ShapeDtypeStruct((n, c_dim), jnp.float32),
            jax.ShapeDtypeStruct((n, c_dim), jnp.float32),
        ],
        compiler_params=pltpu.CompilerParams(
            dimension_semantics=("arbitrary",)),
    )(x, W1, b1_2d, W2, b2_2d, adj)

    ti, tk, fs, fe = [], [], [], []
    for i in range(i0, nb - 1):
        pb = 2 * ((i + 1) // 2)
        ks = list(range(pb // 2, nk))
        for j, k in enumerate(ks):
            ti.append(i)
            tk.append(k)
            fs.append(1 if j == 0 else 0)
            fe.append(1 if k == ks[-1] else 0)
    ti_t = jnp.asarray(np.array(ti, dtype=np.int32))
    tk_t = jnp.asarray(np.array(tk, dtype=np.int32))
    fs_t = jnp.asarray(np.array(fs, dtype=np.int32))
    fe_t = jnp.asarray(np.array(fe, dtype=np.int32))
    n_tiles = ti_t.shape[0]

    out2 = pl.pallas_call(
        functools.partial(_tile_kernel, wb=wb, i0=i0),
        grid_spec=pltpu.PrefetchScalarGridSpec(
            num_scalar_prefetch=4,
            grid=(n_tiles,),
            in_specs=[
                pl.BlockSpec((bm, wb),
                             lambda t, ti, tk, fs, fe: (ti[t], tk[t])),
                pl.BlockSpec((nb * bm, c_dim),
                             lambda t, ti, tk, fs, fe: (0, 0)),
                pl.BlockSpec((bm, c_dim),
                             lambda t, ti, tk, fs, fe: (ti[t], 0)),
            ],
            out_specs=pl.BlockSpec((bm, c_dim),
                                   lambda t, ti, tk, fs, fe: (ti[t] - i0, 0)),
        ),
        out_shape=jax.ShapeDtypeStruct(((nb - 1 - i0) * bm, c_dim),
                                       jnp.float32),
        compiler_params=pltpu.CompilerParams(
            dimension_semantics=("arbitrary",)),
    )(ti_t, tk_t, fs_t, fe_t, adj, s2_hbm, partial)

    lo = i0 * bm
    hi = (nb - 1) * bm
    return jnp.concatenate([out1[:lo], out2, out1[hi:n]], axis=0)


# speedup vs baseline: 1.6493x; 1.0034x over previous
import functools

import jax
import jax.numpy as jnp
import numpy as np
from jax.experimental import pallas as pl
from jax.experimental.pallas import tpu as pltpu


def _log_softmax(logits):
    m = jnp.max(logits, axis=1, keepdims=True)
    z = logits - m
    lse = jnp.log(jnp.sum(jnp.exp(z), axis=1, keepdims=True))
    return z - lse


def _phase_a_kernel(x_ref, w1_ref, b1_ref, w2_ref, b2_ref, adj_ref,
                    s2_out_ref, part_ref, sup_s, s2_s, pend_s,
                    *, bm, nb, n, gb, pmax):
    i = pl.program_id(0)

    @pl.when(i == 0)
    def _():
        sup_s[...] = jnp.dot(x_ref[...], w1_ref[...],
                             preferred_element_type=jnp.float32)
        s2_s[...] = jnp.zeros_like(s2_s)

    h = jnp.dot(adj_ref[...], sup_s[...],
                preferred_element_type=jnp.float32)
    h = jnp.maximum(h + b1_ref[...], 0.0)
    s2 = jnp.dot(h, w2_ref[...], preferred_element_type=jnp.float32)
    row = jax.lax.broadcasted_iota(jnp.int32, s2.shape, 0)
    s2 = jnp.where(row < n - i * bm, s2, 0.0)

    pend_s[pl.ds(jax.lax.rem(i, gb) * bm, bm), :] = s2

    @pl.when((jax.lax.rem(i, gb) == gb - 1) & (i < pmax * gb))
    def _():
        s2_s[pl.ds((i - (gb - 1)) * bm, gb * bm), :] = pend_s[...]

    if pmax > 0:
        @pl.when(i >= gb)
        def _():
            part_ref[...] = jnp.dot(
                adj_ref[:, pl.ds(0, pmax * gb * bm)],
                s2_s[pl.ds(0, pmax * gb * bm), :],
                preferred_element_type=jnp.float32) + b2_ref[...]

    s2_out_ref[...] = s2


def _chunk_kernel(ri_ref, ci_ref, fi_ref, fe_ref, adj_ref, s2_ref,
                  part_ref, b2_ref, o_ref):
    t = pl.program_id(0)
    contrib = jnp.dot(adj_ref[...], s2_ref[...],
                      preferred_element_type=jnp.float32)

    @pl.when(fi_ref[t] == 1)
    def _():
        o_ref[...] = b2_ref[...] + contrib

    @pl.when(fi_ref[t] == 2)
    def _():
        o_ref[...] = part_ref[...] + contrib

    @pl.when(fi_ref[t] == 0)
    def _():
        o_ref[...] += contrib

    @pl.when(fe_ref[t] == 1)
    def _():
        o_ref[...] = _log_softmax(o_ref[...])


def kernel(x, adj, W1, b1, W2, b2):
    n, f_in = x.shape
    h_dim = W1.shape[1]
    c_dim = W2.shape[1]
    bm = 512
    gb = 5
    nb = -(-n // bm)
    wc = gb * bm
    nc = -(-n // wc)
    pmax = nc - 1

    b1_2d = b1.reshape(1, h_dim)
    b2_2d = b2.reshape(1, c_dim)

    s2_hbm, partial = pl.pallas_call(
        functools.partial(_phase_a_kernel, bm=bm, nb=nb, n=n, gb=gb,
                          pmax=pmax),
        grid=(nb,),
        in_specs=[
            pl.BlockSpec((n, f_in), lambda i: (0, 0)),
            pl.BlockSpec((f_in, h_dim), lambda i: (0, 0)),
            pl.BlockSpec((1, h_dim), lambda i: (0, 0)),
            pl.BlockSpec((h_dim, c_dim), lambda i: (0, 0)),
            pl.BlockSpec((1, c_dim), lambda i: (0, 0)),
            pl.BlockSpec((bm, n), lambda i: (i, 0)),
        ],
        out_specs=[
            pl.BlockSpec((bm, c_dim), lambda i: (i, 0)),
            pl.BlockSpec((bm, c_dim), lambda i: (i, 0)),
        ],
        scratch_shapes=[
            pltpu.VMEM((n, h_dim), jnp.float32),
            pltpu.VMEM((nb * bm, c_dim), jnp.float32),
            pltpu.VMEM((gb * bm, c_dim), jnp.float32),
        ],
        out_shape=[
            jax.ShapeDtypeStruct((nb * bm, c_dim), jnp.float32),
            jax.ShapeDtypeStruct((n, c_dim), jnp.float32),
        ],
        compiler_params=pltpu.CompilerParams(
            dimension_semantics=("arbitrary",)),
    )(x, W1, b1_2d, W2, b2_2d, adj)

    ri, ci, fi, fe = [], [], [], []
    for i in range(nb):
        q0 = min(i // gb, pmax)
        for j, c in enumerate(range(q0, nc)):
            ri.append(i)
            ci.append(c)
            fi.append((1 if q0 == 0 else 2) if j == 0 else 0)
            fe.append(1 if c == nc - 1 else 0)
    ri_t = jnp.asarray(np.array(ri, dtype=np.int32))
    ci_t = jnp.asarray(np.array(ci, dtype=np.int32))
    fi_t = jnp.asarray(np.array(fi, dtype=np.int32))
    fe_t = jnp.asarray(np.array(fe, dtype=np.int32))
    n_steps = ri_t.shape[0]

    out = pl.pallas_call(
        _chunk_kernel,
        grid_spec=pltpu.PrefetchScalarGridSpec(
            num_scalar_prefetch=4,
            grid=(n_steps,),
            in_specs=[
                pl.BlockSpec((bm, wc),
                             lambda t, ri, ci, fi, fe: (ri[t], ci[t])),
                pl.BlockSpec((wc, c_dim),
                             lambda t, ri, ci, fi, fe: (ci[t], 0)),
                pl.BlockSpec((bm, c_dim),
                             lambda t, ri, ci, fi, fe: (ri[t], 0)),
                pl.BlockSpec((1, c_dim), lambda t, ri, ci, fi, fe: (0, 0)),
            ],
            out_specs=pl.BlockSpec((bm, c_dim),
                                   lambda t, ri, ci, fi, fe: (ri[t], 0)),
        ),
        out_shape=jax.ShapeDtypeStruct((n, c_dim), jnp.float32),
        compiler_params=pltpu.CompilerParams(
            dimension_semantics=("arbitrary",)),
    )(ri_t, ci_t, fi_t, fe_t, adj, s2_hbm, partial, b2_2d)

    return out


# speedup vs baseline: 1.6656x; 1.0099x over previous
import functools

import jax
import jax.numpy as jnp
import numpy as np
from jax.experimental import pallas as pl
from jax.experimental.pallas import tpu as pltpu


def _log_softmax(logits):
    m = jnp.max(logits, axis=1, keepdims=True)
    z = logits - m
    lse = jnp.log(jnp.sum(jnp.exp(z), axis=1, keepdims=True))
    return z - lse


def _phase_a_kernel(x_ref, w1_ref, b1_ref, w2_ref, b2_ref, adj_ref,
                    s2_out_ref, part_ref, sup_s, s2_s, pend_s,
                    *, bm, nb, n, gb, pmax):
    i = pl.program_id(0)

    @pl.when(i == 0)
    def _():
        sup_s[...] = jnp.dot(x_ref[...], w1_ref[...],
                             preferred_element_type=jnp.float32)
        s2_s[...] = jnp.zeros_like(s2_s)

    h = jnp.dot(adj_ref[...], sup_s[...],
                preferred_element_type=jnp.float32)
    h = jnp.maximum(h + b1_ref[...], 0.0)
    s2 = jnp.dot(h, w2_ref[...], preferred_element_type=jnp.float32)
    row = jax.lax.broadcasted_iota(jnp.int32, s2.shape, 0)
    s2 = jnp.where(row < n - i * bm, s2, 0.0)

    if pmax > 0:
        @pl.when(i >= gb)
        def _():
            part_ref[...] = jnp.dot(
                adj_ref[:, pl.ds(0, pmax * gb * bm)],
                s2_s[pl.ds(0, pmax * gb * bm), :],
                preferred_element_type=jnp.float32) + b2_ref[...]

    pend_s[pl.ds(jax.lax.rem(i, gb) * bm, bm), :] = s2

    @pl.when((jax.lax.rem(i, gb) == gb - 1) & (i < pmax * gb))
    def _():
        s2_s[pl.ds((i - (gb - 1)) * bm, gb * bm), :] = pend_s[...]

    s2_out_ref[...] = s2


def _chunk_kernel(ri_ref, ci_ref, fi_ref, fe_ref, adj_ref, s2_ref,
                  part_ref, b2_ref, o_ref):
    t = pl.program_id(0)
    contrib = jnp.dot(adj_ref[...], s2_ref[...],
                      preferred_element_type=jnp.float32)

    @pl.when(fi_ref[t] == 1)
    def _():
        o_ref[...] = b2_ref[...] + contrib

    @pl.when(fi_ref[t] == 2)
    def _():
        o_ref[...] = part_ref[...] + contrib

    @pl.when(fi_ref[t] == 0)
    def _():
        o_ref[...] += contrib

    @pl.when(fe_ref[t] == 1)
    def _():
        o_ref[...] = _log_softmax(o_ref[...])


def kernel(x, adj, W1, b1, W2, b2):
    n, f_in = x.shape
    h_dim = W1.shape[1]
    c_dim = W2.shape[1]
    bm = 512
    gb = 5
    nb = -(-n // bm)
    wc = gb * bm
    nc = -(-n // wc)
    pmax = nc - 1

    b1_2d = b1.reshape(1, h_dim)
    b2_2d = b2.reshape(1, c_dim)

    s2_hbm, partial = pl.pallas_call(
        functools.partial(_phase_a_kernel, bm=bm, nb=nb, n=n, gb=gb,
                          pmax=pmax),
        grid=(nb,),
        in_specs=[
            pl.BlockSpec((n, f_in), lambda i: (0, 0)),
            pl.BlockSpec((f_in, h_dim), lambda i: (0, 0)),
            pl.BlockSpec((1, h_dim), lambda i: (0, 0)),
            pl.BlockSpec((h_dim, c_dim), lambda i: (0, 0)),
            pl.BlockSpec((1, c_dim), lambda i: (0, 0)),
            pl.BlockSpec((bm, n), lambda i: (i, 0)),
        ],
        out_specs=[
            pl.BlockSpec((bm, c_dim), lambda i: (i, 0)),
            pl.BlockSpec((bm, c_dim), lambda i: (i, 0)),
        ],
        scratch_shapes=[
            pltpu.VMEM((n, h_dim), jnp.float32),
            pltpu.VMEM((nb * bm, c_dim), jnp.float32),
            pltpu.VMEM((gb * bm, c_dim), jnp.float32),
        ],
        out_shape=[
            jax.ShapeDtypeStruct((nb * bm, c_dim), jnp.float32),
            jax.ShapeDtypeStruct((n, c_dim), jnp.float32),
        ],
        compiler_params=pltpu.CompilerParams(
            dimension_semantics=("arbitrary",)),
    )(x, W1, b1_2d, W2, b2_2d, adj)

    ri, ci, fi, fe = [], [], [], []
    for i in range(nb):
        q0 = min(i // gb, pmax)
        for j, c in enumerate(range(q0, nc)):
            ri.append(i)
            ci.append(c)
            fi.append((1 if q0 == 0 else 2) if j == 0 else 0)
            fe.append(1 if c == nc - 1 else 0)
    ri_t = jnp.asarray(np.array(ri, dtype=np.int32))
    ci_t = jnp.asarray(np.array(ci, dtype=np.int32))
    fi_t = jnp.asarray(np.array(fi, dtype=np.int32))
    fe_t = jnp.asarray(np.array(fe, dtype=np.int32))
    n_steps = ri_t.shape[0]

    out = pl.pallas_call(
        _chunk_kernel,
        grid_spec=pltpu.PrefetchScalarGridSpec(
            num_scalar_prefetch=4,
            grid=(n_steps,),
            in_specs=[
                pl.BlockSpec((bm, wc),
                             lambda t, ri, ci, fi, fe: (ri[t], ci[t])),
                pl.BlockSpec((wc, c_dim),
                             lambda t, ri, ci, fi, fe: (ci[t], 0)),
                pl.BlockSpec((bm, c_dim),
                             lambda t, ri, ci, fi, fe: (ri[t], 0)),
                pl.BlockSpec((1, c_dim), lambda t, ri, ci, fi, fe: (0, 0)),
            ],
            out_specs=pl.BlockSpec((bm, c_dim),
                                   lambda t, ri, ci, fi, fe: (ri[t], 0)),
        ),
        out_shape=jax.ShapeDtypeStruct((n, c_dim), jnp.float32),
        compiler_params=pltpu.CompilerParams(
            dimension_semantics=("arbitrary",)),
    )(ri_t, ci_t, fi_t, fe_t, adj, s2_hbm, partial, b2_2d)

    return out
